# Initial kernel scaffold; baseline (speedup 1.0000x reference)
import jax
import jax.numpy as jnp
from jax.experimental import pallas as pl


def kernel(sequence, segment_label, token_table, segment_table, pos_emb):
    raise NotImplementedError("write your pallas kernel here")


# speedup vs baseline: 2.4935x; 2.4935x over previous
import functools

import jax
import jax.numpy as jnp
from jax import lax
from jax.experimental import pallas as pl
from jax.experimental.pallas import tpu as pltpu
from jax.experimental.pallas import tpu_sc as plsc

D = 128
G = 128
NC, NS = 2, 16
NW = NC * NS


def _sc_gather(idx2d, table):
    n_groups = idx2d.shape[0]
    g_per_w = n_groups // NW
    mesh = plsc.VectorSubcoreMesh(core_axis_name="c", subcore_axis_name="s")

    @functools.partial(
        pl.kernel,
        mesh=mesh,
        out_type=jax.ShapeDtypeStruct((n_groups, G, D), jnp.float32),
        scratch_types=[
            pltpu.VMEM((g_per_w, G), jnp.int32),
            pltpu.VMEM((G, D), jnp.float32),
            pltpu.SemaphoreType.DMA,
        ],
    )
    def k(idx_hbm, table_hbm, out_hbm, idx_v, rows_v, sem_g):
        wid = lax.axis_index("s") * NC + lax.axis_index("c")
        base = wid * g_per_w
        pltpu.sync_copy(idx_hbm.at[pl.ds(base, g_per_w)], idx_v)

        def body(g, carry):
            pltpu.async_copy(table_hbm.at[idx_v.at[g]], rows_v, sem_g).wait()
            pltpu.sync_copy(rows_v, out_hbm.at[base + g])
            return carry

        lax.fori_loop(0, g_per_w, body, 0)

    return k(idx2d, table)


def _tc_add_body(tok_ref, lab_ref, pos_ref, seg_ref, out_ref):
    lab = lab_ref[0, 0, :][:, None]
    seg = jnp.where(
        lab == 1, seg_ref[1, :][None, :],
        jnp.where(lab == 2, seg_ref[2, :][None, :], seg_ref[0, :][None, :]))
    out_ref[...] = tok_ref[...] + pos_ref[...] + seg


def _tc_add(tok_flat, labels3d, pos2d, seg_table):
    n = tok_flat.shape[0]
    R = 1024
    nblocks = n // R
    blocks_per_l = pos2d.shape[0] // R
    return pl.pallas_call(
        _tc_add_body,
        grid=(nblocks,),
        in_specs=[
            pl.BlockSpec((R, D), lambda i: (i, 0)),
            pl.BlockSpec((1, 1, R), lambda i: (i, 0, 0)),
            pl.BlockSpec((R, D), lambda i: (i % blocks_per_l, 0)),
            pl.BlockSpec((3, D), lambda i: (0, 0)),
        ],
        out_specs=pl.BlockSpec((R, D), lambda i: (i, 0)),
        out_shape=jax.ShapeDtypeStruct((n, D), jnp.float32),
    )(tok_flat, labels3d, pos2d, seg_table)


def kernel(sequence, segment_label, token_table, segment_table, pos_emb):
    B, L = sequence.shape
    N = B * L
    idx2d = sequence.reshape(N // G, G)
    tok = _sc_gather(idx2d, token_table)
    tok_flat = tok.reshape(N, D)
    labels3d = segment_label.reshape(N // 1024, 1, 1024)
    pos2d = pos_emb.reshape(L, D)
    out_flat = _tc_add(tok_flat, labels3d, pos2d, segment_table)
    return out_flat.reshape(B, L, D)


# speedup vs baseline: 2.7558x; 1.1052x over previous
import functools

import jax
import jax.numpy as jnp
from jax import lax
from jax.experimental import pallas as pl
from jax.experimental.pallas import tpu as pltpu
from jax.experimental.pallas import tpu_sc as plsc

D = 128
G = 128
NC, NS = 2, 16
NW = NC * NS


def _sc_gather(idx2d, table):
    n_groups = idx2d.shape[0]
    g_per_w = n_groups // NW
    mesh = plsc.VectorSubcoreMesh(core_axis_name="c", subcore_axis_name="s")

    NBUF = 4
    n_it = g_per_w // NBUF

    @functools.partial(
        pl.kernel,
        mesh=mesh,
        out_type=jax.ShapeDtypeStruct((n_groups, G, D), jnp.float32),
        scratch_types=[
            pltpu.VMEM((g_per_w, G), jnp.int32),
        ]
        + [pltpu.VMEM((G, D), jnp.float32) for _ in range(NBUF)]
        + [pltpu.SemaphoreType.DMA for _ in range(2 * NBUF)],
    )
    def k(idx_hbm, table_hbm, out_hbm, idx_v, *scratch):
        bufs = scratch[:NBUF]
        sg = scratch[NBUF:2 * NBUF]
        so = scratch[2 * NBUF:]
        wid = lax.axis_index("s") * NC + lax.axis_index("c")
        base = wid * g_per_w
        pltpu.sync_copy(idx_hbm.at[pl.ds(base, g_per_w)], idx_v)

        for b in range(NBUF):
            pltpu.async_copy(table_hbm.at[idx_v.at[b]], bufs[b], sg[b])

        def body(it, carry):
            g0 = it * NBUF
            for b in range(NBUF):
                pltpu.make_async_copy(table_hbm.at[idx_v.at[0]], bufs[b], sg[b]).wait()
                pltpu.async_copy(bufs[b], out_hbm.at[base + g0 + b], so[b])

            @pl.when(it < n_it - 1)
            def _prefetch():
                for b in range(NBUF):
                    pltpu.make_async_copy(bufs[b], out_hbm.at[base], so[b]).wait()
                    pltpu.async_copy(table_hbm.at[idx_v.at[g0 + NBUF + b]], bufs[b], sg[b])

            return carry

        lax.fori_loop(0, n_it, body, 0)
        for b in range(NBUF):
            pltpu.make_async_copy(bufs[b], out_hbm.at[base], so[b]).wait()

    return k(idx2d, table)


def _tc_add_body(tok_ref, lab_ref, pos_ref, seg_ref, out_ref):
    lab = lab_ref[0, 0, :][:, None]
    seg = jnp.where(
        lab == 1, seg_ref[1, :][None, :],
        jnp.where(lab == 2, seg_ref[2, :][None, :], seg_ref[0, :][None, :]))
    out_ref[...] = tok_ref[...] + pos_ref[...] + seg


def _tc_add(tok_flat, labels3d, pos2d, seg_table):
    n = tok_flat.shape[0]
    R = 1024
    nblocks = n // R
    blocks_per_l = pos2d.shape[0] // R
    return pl.pallas_call(
        _tc_add_body,
        grid=(nblocks,),
        in_specs=[
            pl.BlockSpec((R, D), lambda i: (i, 0)),
            pl.BlockSpec((1, 1, R), lambda i: (i, 0, 0)),
            pl.BlockSpec((R, D), lambda i: (i % blocks_per_l, 0)),
            pl.BlockSpec((3, D), lambda i: (0, 0)),
        ],
        out_specs=pl.BlockSpec((R, D), lambda i: (i, 0)),
        out_shape=jax.ShapeDtypeStruct((n, D), jnp.float32),
    )(tok_flat, labels3d, pos2d, seg_table)


def kernel(sequence, segment_label, token_table, segment_table, pos_emb):
    B, L = sequence.shape
    N = B * L
    idx2d = sequence.reshape(N // G, G)
    tok = _sc_gather(idx2d, token_table)
    tok_flat = tok.reshape(N, D)
    labels3d = segment_label.reshape(N // 1024, 1, 1024)
    pos2d = pos_emb.reshape(L, D)
    out_flat = _tc_add(tok_flat, labels3d, pos2d, segment_table)
    return out_flat.reshape(B, L, D)


# speedup vs baseline: 3.0215x; 1.0964x over previous
import functools

import jax
import jax.numpy as jnp
from jax import lax
from jax.experimental import pallas as pl
from jax.experimental.pallas import tpu as pltpu
from jax.experimental.pallas import tpu_sc as plsc

D = 128
G = 128
NC, NS = 2, 16
NW = NC * NS


def _sc_gather(idx2d, table):
    n_groups = idx2d.shape[0]
    g_per_w = n_groups // NW
    mesh = plsc.VectorSubcoreMesh(core_axis_name="c", subcore_axis_name="s")

    NBUF = 4
    n_it = g_per_w // NBUF

    @functools.partial(
        pl.kernel,
        mesh=mesh,
        out_type=jax.ShapeDtypeStruct((n_groups, G, D), jnp.float32),
        scratch_types=[
            pltpu.VMEM((g_per_w, G), jnp.int32),
        ]
        + [pltpu.VMEM((G, D), jnp.float32) for _ in range(NBUF)]
        + [pltpu.SemaphoreType.DMA for _ in range(2 * NBUF)],
    )
    def k(idx_hbm, table_hbm, out_hbm, idx_v, *scratch):
        bufs = scratch[:NBUF]
        sg = scratch[NBUF:2 * NBUF]
        so = scratch[2 * NBUF:]
        wid = lax.axis_index("s") * NC + lax.axis_index("c")
        base = wid * g_per_w
        pltpu.sync_copy(idx_hbm.at[pl.ds(base, g_per_w)], idx_v)

        for b in range(NBUF):
            pltpu.async_copy(table_hbm.at[idx_v.at[b]], bufs[b], sg[b])

        def body(it, carry):
            g0 = it * NBUF
            for b in range(NBUF):
                pltpu.make_async_copy(table_hbm.at[idx_v.at[0]], bufs[b], sg[b]).wait()
                pltpu.async_copy(bufs[b], out_hbm.at[base + g0 + b], so[b])

            @pl.when(it < n_it - 1)
            def _prefetch():
                for b in range(NBUF):
                    pltpu.make_async_copy(bufs[b], out_hbm.at[base], so[b]).wait()
                    pltpu.async_copy(table_hbm.at[idx_v.at[g0 + NBUF + b]], bufs[b], sg[b])

            return carry

        lax.fori_loop(0, n_it, body, 0)
        for b in range(NBUF):
            pltpu.make_async_copy(bufs[b], out_hbm.at[base], so[b]).wait()

    return k(idx2d, table)


def _tc_add_body(tok_ref, lab_ref, pos_ref, seg_ref, out_ref):
    lab = lab_ref[0, 0, :][:, None]
    seg = jnp.where(
        lab == 1, seg_ref[1, :][None, :],
        jnp.where(lab == 2, seg_ref[2, :][None, :], seg_ref[0, :][None, :]))
    out_ref[...] = tok_ref[...] + pos_ref[...] + seg


def _tc_add(tok_flat, labels3d, pos2d, seg_table):
    n = tok_flat.shape[0]
    R = 1024
    nblocks = n // R
    blocks_per_l = pos2d.shape[0] // R
    nbatch = nblocks // blocks_per_l
    return pl.pallas_call(
        _tc_add_body,
        grid=(blocks_per_l, nbatch),
        in_specs=[
            pl.BlockSpec((R, D), lambda lc, b: (b * blocks_per_l + lc, 0)),
            pl.BlockSpec((1, 1, R), lambda lc, b: (b * blocks_per_l + lc, 0, 0)),
            pl.BlockSpec((R, D), lambda lc, b: (lc, 0)),
            pl.BlockSpec((3, D), lambda lc, b: (0, 0)),
        ],
        out_specs=pl.BlockSpec((R, D), lambda lc, b: (b * blocks_per_l + lc, 0)),
        out_shape=jax.ShapeDtypeStruct((n, D), jnp.float32),
    )(tok_flat, labels3d, pos2d, seg_table)


def kernel(sequence, segment_label, token_table, segment_table, pos_emb):
    B, L = sequence.shape
    N = B * L
    idx2d = sequence.reshape(N // G, G)
    tok = _sc_gather(idx2d, token_table)
    tok_flat = tok.reshape(N, D)
    labels3d = segment_label.reshape(N // 1024, 1, 1024)
    pos2d = pos_emb.reshape(L, D)
    out_flat = _tc_add(tok_flat, labels3d, pos2d, segment_table)
    return out_flat.reshape(B, L, D)


# speedup vs baseline: 5.2081x; 1.7237x over previous
import functools

import jax
import jax.numpy as jnp
from jax import lax
from jax.experimental import pallas as pl
from jax.experimental.pallas import tpu as pltpu
from jax.experimental.pallas import tpu_sc as plsc

D = 128
G = 128
NC, NS = 2, 16
NW = NC * NS
NBUF = 2


def _make_base(pos2d, seg_table):
    L = pos2d.shape[0]
    S = seg_table.shape[0]

    def body(pos_ref, seg_ref, out_ref):
        s = pl.program_id(0)
        out_ref[...] = pos_ref[...] + seg_ref[pl.ds(s, 1), :]

    return pl.pallas_call(
        body,
        grid=(S,),
        in_specs=[
            pl.BlockSpec((L, D), lambda s: (0, 0)),
            pl.BlockSpec((S, D), lambda s: (0, 0)),
        ],
        out_specs=pl.BlockSpec((L, D), lambda s: (s, 0)),
        out_shape=jax.ShapeDtypeStruct((S * L, D), jnp.float32),
    )(pos2d, seg_table)


def _sc_fused(idx2d, lab2d, table, base, L):
    n_groups = idx2d.shape[0]
    g_per_w = n_groups // NW
    n_it = g_per_w // NBUF
    gpl = L // G
    mesh = plsc.VectorSubcoreMesh(core_axis_name="c", subcore_axis_name="s")

    @functools.partial(
        pl.kernel,
        mesh=mesh,
        out_type=jax.ShapeDtypeStruct((n_groups, G, D), jnp.float32),
        scratch_types=[
            pltpu.VMEM((g_per_w, G), jnp.int32),
            pltpu.VMEM((g_per_w, G), jnp.int32),
            pltpu.VMEM_SHARED(base.shape, jnp.float32),
        ]
        + [pltpu.VMEM((G,), jnp.int32) for _ in range(NBUF)]
        + [pltpu.VMEM((G, D), jnp.float32) for _ in range(2 * NBUF)]
        + [pltpu.SemaphoreType.DMA for _ in range(3 * NBUF)],
    )
    def k(idx_hbm, lab_hbm, table_hbm, base_hbm, out_hbm, idx_v, lab_v,
          base_sh, *scratch):
        cidx = scratch[:NBUF]
        tok = scratch[NBUF:2 * NBUF]
        bas = scratch[2 * NBUF:3 * NBUF]
        sgt = scratch[3 * NBUF:4 * NBUF]
        sgb = scratch[4 * NBUF:5 * NBUF]
        so = scratch[5 * NBUF:6 * NBUF]

        wid = lax.axis_index("s") * NC + lax.axis_index("c")
        wbase = wid * g_per_w

        @pl.when(lax.axis_index("s") == 0)
        def _init_base():
            pltpu.sync_copy(base_hbm, base_sh)

        pltpu.sync_copy(idx_hbm.at[pl.ds(wbase, g_per_w)], idx_v)
        pltpu.sync_copy(lab_hbm.at[pl.ds(wbase, g_per_w)], lab_v)
        plsc.subcore_barrier()

        def start_gathers(g, b):
            l0 = lax.rem(wbase + g, gpl) * G
            for c in range(G // 16):
                lab16 = lab_v[g, pl.ds(c * 16, 16)]
                cidx[b][pl.ds(c * 16, 16)] = (
                    lab16 * L + (l0 + c * 16) + lax.iota(jnp.int32, 16))
            pltpu.async_copy(table_hbm.at[idx_v.at[g]], tok[b], sgt[b])
            pltpu.async_copy(base_sh.at[cidx[b]], bas[b], sgb[b])

        for b in range(NBUF):
            start_gathers(b, b)

        def body(it, carry):
            g0 = it * NBUF
            for b in range(NBUF):
                pltpu.make_async_copy(table_hbm.at[idx_v.at[0]], tok[b], sgt[b]).wait()
                pltpu.make_async_copy(base_sh.at[cidx[b]], bas[b], sgb[b]).wait()

                def row(r, c2):
                    for c in range(D // 16):
                        sl = pl.ds(c * 16, 16)
                        tok[b][r, sl] = tok[b][r, sl] + bas[b][r, sl]
                    return c2

                lax.fori_loop(0, G, row, 0)
                pltpu.async_copy(tok[b], out_hbm.at[wbase + g0 + b], so[b])

            @pl.when(it < n_it - 1)
            def _prefetch():
                for b in range(NBUF):
                    pltpu.make_async_copy(tok[b], out_hbm.at[wbase], so[b]).wait()
                    start_gathers(g0 + NBUF + b, b)

            return carry

        lax.fori_loop(0, n_it, body, 0)
        for b in range(NBUF):
            pltpu.make_async_copy(tok[b], out_hbm.at[wbase], so[b]).wait()

    return k(idx2d, lab2d, table, base)


def kernel(sequence, segment_label, token_table, segment_table, pos_emb):
    B, L = sequence.shape
    N = B * L
    idx2d = sequence.reshape(N // G, G)
    lab2d = segment_label.reshape(N // G, G)
    pos2d = pos_emb.reshape(L, D)
    base = _make_base(pos2d, segment_table)
    out = _sc_fused(idx2d, lab2d, token_table, base, L)
    return out.reshape(B, L, D)


# speedup vs baseline: 5.3179x; 1.0211x over previous
import functools

import jax
import jax.numpy as jnp
from jax import lax
from jax.experimental import pallas as pl
from jax.experimental.pallas import tpu as pltpu
from jax.experimental.pallas import tpu_sc as plsc

D = 128
G = 128
NC, NS = 2, 16
NW = NC * NS
NT = 2
NB = 2


def _sc_fused(idx2d, lab2d, table, pos2d, seg_table, L):
    n_groups = idx2d.shape[0]
    g_per_w = n_groups // NW
    gpl = L // G
    lpt = L // NS
    S = seg_table.shape[0]
    mesh = plsc.VectorSubcoreMesh(core_axis_name="c", subcore_axis_name="s")

    @functools.partial(
        pl.kernel,
        mesh=mesh,
        out_type=jax.ShapeDtypeStruct((n_groups * G, D), jnp.float32),
        scratch_types=[
            pltpu.VMEM((g_per_w, G), jnp.int32),
            pltpu.VMEM((g_per_w, G), jnp.int32),
            pltpu.VMEM((S, D), jnp.float32),
            pltpu.VMEM_SHARED((S * L, D), jnp.float32),
        ]
        + [pltpu.VMEM((G,), jnp.int32) for _ in range(2 * NB)]
        + [pltpu.VMEM((G, D), jnp.float32) for _ in range(NT + NB)]
        + [pltpu.SemaphoreType.DMA for _ in range(NT + 2 * NB)],
    )
    def k(idx_hbm, lab_hbm, table_hbm, pos_hbm, seg_hbm, out_hbm, idx_v,
          lab_v, seg_v, base_sh, *scratch):
        cidx = scratch[:NB]
        oidx = scratch[NB:2 * NB]
        tok = scratch[2 * NB:2 * NB + NT]
        bas = scratch[2 * NB + NT:2 * NB + NT + NB]
        sgt = scratch[2 * NB + NT + NB:2 * NB + 2 * NT + NB]
        sgb = scratch[2 * NB + 2 * NT + NB:2 * NB + 2 * NT + 2 * NB]
        so = scratch[2 * NB + 2 * NT + 2 * NB:]

        sid = lax.axis_index("s")
        wid = sid * NC + lax.axis_index("c")
        wbase = wid * g_per_w

        pltpu.sync_copy(idx_hbm.at[pl.ds(wbase, g_per_w)], idx_v)
        pltpu.sync_copy(lab_hbm.at[pl.ds(wbase, g_per_w)], lab_v)

        for b in range(NT):
            pltpu.async_copy(table_hbm.at[idx_v.at[b]], tok[b], sgt[b])

        pltpu.sync_copy(seg_hbm, seg_v)
        for c in range(lpt // 16):
            cidx[0][pl.ds(c * 16, 16)] = (
                sid * lpt + c * 16 + lax.iota(jnp.int32, 16))
        pltpu.async_copy(pos_hbm.at[cidx[0]], bas[0], sgb[0]).wait()
        for s in range(S):
            if s == 0:
                d16 = [seg_v[0, pl.ds(c * 16, 16)] for c in range(D // 16)]
            else:
                d16 = [seg_v[s, pl.ds(c * 16, 16)]
                       - seg_v[s - 1, pl.ds(c * 16, 16)]
                       for c in range(D // 16)]

            def brow(r, c2, d16=d16):
                for c in range(D // 16):
                    sl = pl.ds(c * 16, 16)
                    bas[0][r, sl] = bas[0][r, sl] + d16[c]
                return c2

            lax.fori_loop(0, lpt, brow, 0)
            pltpu.sync_copy(bas[0], base_sh.at[pl.ds(s * L + sid * lpt, lpt)])
        plsc.subcore_barrier()

        def comp_cidx(g, b):
            l0 = lax.rem(wbase + g, gpl) * G
            for c in range(G // 16):
                lab16 = lab_v[g, pl.ds(c * 16, 16)]
                cidx[b][pl.ds(c * 16, 16)] = (
                    lab16 * L + (l0 + c * 16) + lax.iota(jnp.int32, 16))

        def prepare(h):
            b = h % NB
            if h >= NB:
                pltpu.make_async_copy(bas[b], out_hbm.at[oidx[b]], so[b]).wait()
            comp_cidx(h, b)
            pltpu.async_copy(base_sh.at[cidx[b]], bas[b], sgb[b])

        def process(g):
            bt = g % NT
            bb = g % NB
            pltpu.make_async_copy(table_hbm.at[idx_v.at[0]], tok[bt], sgt[bt]).wait()
            pltpu.make_async_copy(base_sh.at[cidx[bb]], bas[bb], sgb[bb]).wait()

            def row(r, c2, bt=bt, bb=bb):
                for u in range(2):
                    for c in range(D // 16):
                        sl = pl.ds(c * 16, 16)
                        plsc.addupdate(bas[bb].at[2 * r + u, sl],
                                       tok[bt][2 * r + u, sl])
                return c2

            lax.fori_loop(0, G // 2, row, 0)
            if g + NT < g_per_w:
                pltpu.async_copy(table_hbm.at[idx_v.at[g + NT]], tok[bt], sgt[bt])
            r0 = (wbase + g) * G
            for c in range(G // 16):
                oidx[bb][pl.ds(c * 16, 16)] = (
                    r0 + c * 16 + lax.iota(jnp.int32, 16))
            pltpu.async_copy(bas[bb], out_hbm.at[oidx[bb]], so[bb])

        for h in range(NB):
            prepare(h)
        for g in range(g_per_w):
            if NB <= g + 1 < g_per_w:
                prepare(g + 1)
            process(g)
        for b in range(NB):
            pltpu.make_async_copy(bas[b], out_hbm.at[oidx[b]], so[b]).wait()

    return k(idx2d, lab2d, table, pos2d, seg_table)


def kernel(sequence, segment_label, token_table, segment_table, pos_emb):
    B, L = sequence.shape
    N = B * L
    idx2d = sequence.reshape(N // G, G)
    lab2d = segment_label.reshape(N // G, G)
    pos2d = pos_emb.reshape(L, D)
    out = _sc_fused(idx2d, lab2d, token_table, pos2d, segment_table, L)
    return out.reshape(B, L, D)


# speedup vs baseline: 5.4560x; 1.0260x over previous
import functools

import jax
import jax.numpy as jnp
from jax import lax
from jax.experimental import pallas as pl
from jax.experimental.pallas import tpu as pltpu
from jax.experimental.pallas import tpu_sc as plsc

D = 128
G = 128
NC, NS = 2, 16
NW = NC * NS
NT = 2
NB = 2


def _sc_fused(idx2d, lab2d, table, pos2d, seg_table, L):
    n_groups = idx2d.shape[0]
    g_per_w = n_groups // NW
    gpl = L // G
    lpt = L // NS
    S = seg_table.shape[0]
    mesh = plsc.VectorSubcoreMesh(core_axis_name="c", subcore_axis_name="s")

    @functools.partial(
        pl.kernel,
        mesh=mesh,
        out_type=jax.ShapeDtypeStruct((n_groups * G, D), jnp.float32),
        scratch_types=[
            pltpu.VMEM((g_per_w, G), jnp.int32),
            pltpu.VMEM((g_per_w, G), jnp.int32),
            pltpu.VMEM((S, D), jnp.float32),
            pltpu.VMEM_SHARED((S * L, D), jnp.float32),
        ]
        + [pltpu.VMEM((G,), jnp.int32) for _ in range(NB)]
        + [pltpu.VMEM((G, D), jnp.float32) for _ in range(NT + NB)]
        + [pltpu.SemaphoreType.DMA for _ in range(NT + 2 * NB)],
    )
    def k(idx_hbm, lab_hbm, table_hbm, pos_hbm, seg_hbm, out_hbm, idx_v,
          lab_v, seg_v, base_sh, *scratch):
        cidx = scratch[:NB]
        tok = scratch[NB:NB + NT]
        bas = scratch[NB + NT:NB + NT + NB]
        sgt = scratch[NB + NT + NB:NB + 2 * NT + NB]
        sgb = scratch[NB + 2 * NT + NB:NB + 2 * NT + 2 * NB]
        so = scratch[NB + 2 * NT + 2 * NB:]

        sid = lax.axis_index("s")
        wid = sid * NC + lax.axis_index("c")
        wbase = wid * g_per_w

        pltpu.sync_copy(idx_hbm.at[pl.ds(wbase, g_per_w)], idx_v)
        pltpu.sync_copy(lab_hbm.at[pl.ds(wbase, g_per_w)], lab_v)

        for b in range(NT):
            pltpu.async_copy(table_hbm.at[idx_v.at[b]], tok[b], sgt[b])

        pltpu.sync_copy(seg_hbm, seg_v)
        for c in range(lpt // 16):
            cidx[0][pl.ds(c * 16, 16)] = (
                sid * lpt + c * 16 + lax.iota(jnp.int32, 16))
        pltpu.async_copy(pos_hbm.at[cidx[0]], bas[0], sgb[0]).wait()
        for s in range(S):
            if s == 0:
                d16 = [seg_v[0, pl.ds(c * 16, 16)] for c in range(D // 16)]
            else:
                d16 = [seg_v[s, pl.ds(c * 16, 16)]
                       - seg_v[s - 1, pl.ds(c * 16, 16)]
                       for c in range(D // 16)]

            def brow(r, c2, d16=d16):
                for c in range(D // 16):
                    sl = pl.ds(c * 16, 16)
                    bas[0][r, sl] = bas[0][r, sl] + d16[c]
                return c2

            lax.fori_loop(0, lpt, brow, 0)
            pltpu.sync_copy(bas[0], base_sh.at[pl.ds(s * L + sid * lpt, lpt)])
        plsc.subcore_barrier()

        def comp_cidx(g, b):
            l0 = lax.rem(wbase + g, gpl) * G
            for c in range(G // 16):
                lab16 = lab_v[g, pl.ds(c * 16, 16)]
                cidx[b][pl.ds(c * 16, 16)] = (
                    lab16 * L + (l0 + c * 16) + lax.iota(jnp.int32, 16))

        def prepare(h):
            b = h % NB
            if h >= NB:
                pltpu.make_async_copy(bas[b], out_hbm.at[pl.ds(0, G)], so[b]).wait()
            comp_cidx(h, b)
            pltpu.async_copy(base_sh.at[cidx[b]], bas[b], sgb[b])

        def process(g):
            bt = g % NT
            bb = g % NB
            pltpu.make_async_copy(table_hbm.at[idx_v.at[0]], tok[bt], sgt[bt]).wait()
            pltpu.make_async_copy(base_sh.at[cidx[bb]], bas[bb], sgb[bb]).wait()

            def row(r, c2, bt=bt, bb=bb):
                for u in range(2):
                    for c in range(D // 16):
                        sl = pl.ds(c * 16, 16)
                        plsc.addupdate(bas[bb].at[2 * r + u, sl],
                                       tok[bt][2 * r + u, sl])
                return c2

            lax.fori_loop(0, G // 2, row, 0)
            if g + NT < g_per_w:
                pltpu.async_copy(table_hbm.at[idx_v.at[g + NT]], tok[bt], sgt[bt])
            r0 = (wbase + g) * G
            pltpu.async_copy(bas[bb], out_hbm.at[pl.ds(r0, G)], so[bb])

        for h in range(NB):
            prepare(h)
        for g in range(g_per_w):
            if NB <= g + 1 < g_per_w:
                prepare(g + 1)
            process(g)
        for b in range(NB):
            pltpu.make_async_copy(bas[b], out_hbm.at[pl.ds(0, G)], so[b]).wait()

    return k(idx2d, lab2d, table, pos2d, seg_table)


def kernel(sequence, segment_label, token_table, segment_table, pos_emb):
    B, L = sequence.shape
    N = B * L
    idx2d = sequence.reshape(N // G, G)
    lab2d = segment_label.reshape(N // G, G)
    pos2d = pos_emb.reshape(L, D)
    out = _sc_fused(idx2d, lab2d, token_table, pos2d, segment_table, L)
    return out.reshape(B, L, D)
